# Initial kernel scaffold; baseline (speedup 1.0000x reference)
import jax
import jax.numpy as jnp
from jax.experimental import pallas as pl


def kernel(naics_2_digit, naics_3_digit, naics_4_digit, naics_5_digit, naics_6_digit, table2, delta3, delta4, delta5, delta6, W, b):
    raise NotImplementedError("write your pallas kernel here")


# speedup vs baseline: 7.1499x; 7.1499x over previous
import jax
import jax.numpy as jnp
from jax import lax
from jax.experimental import pallas as pl

_B = 16384
_D = 128
_BLK = 512
_NBLK = _B // _BLK


def _onehot_gather(idx, table, v):
    oh = (idx[:, None] == lax.broadcasted_iota(jnp.int32, (_BLK, v), 1)).astype(jnp.float32)
    return jnp.dot(oh, table, preferred_element_type=jnp.float32)


def _norm(e):
    n = jnp.sum(e * e, axis=1, keepdims=True)
    return e * lax.rsqrt(jnp.maximum(n, 1e-24))


def _body(i2_ref, i3_ref, i4_ref, i5_ref, i6_ref,
          t2_ref, d3_ref, d4_ref, d5_ref, d6_ref, w_ref, b_ref, out_ref):
    e = _norm(_onehot_gather(i2_ref[0, 0, :], t2_ref[...], 25))
    e = _norm(e + _onehot_gather(i3_ref[0, 0, :], d3_ref[...], 100))
    e = _norm(e + _onehot_gather(i4_ref[0, 0, :], d4_ref[...], 400))
    e = _norm(e + _onehot_gather(i5_ref[0, 0, :], d5_ref[...], 700))
    e = _norm(e + _onehot_gather(i6_ref[0, 0, :], d6_ref[...], 1057))
    out_ref[...] = jnp.sum(e * w_ref[...], axis=1, keepdims=True) + b_ref[0, 0]


def kernel(naics_2_digit, naics_3_digit, naics_4_digit, naics_5_digit, naics_6_digit,
           table2, delta3, delta4, delta5, delta6, W, b):
    idxs = [x.reshape(_NBLK, 1, _BLK) for x in
            (naics_2_digit, naics_3_digit, naics_4_digit, naics_5_digit, naics_6_digit)]
    tabs = [table2, delta3, delta4, delta5, delta6]

    idx_spec = pl.BlockSpec((1, 1, _BLK), lambda i: (i, 0, 0))
    full = lambda shp: pl.BlockSpec(shp, lambda i: (0,) * len(shp))

    out = pl.pallas_call(
        _body,
        grid=(_NBLK,),
        in_specs=[idx_spec] * 5 + [full(t.shape) for t in tabs] + [full((1, _D)), full((1, 1))],
        out_specs=pl.BlockSpec((_BLK, 1), lambda i: (i, 0)),
        out_shape=jax.ShapeDtypeStruct((_B, 1), jnp.float32),
    )(*idxs, *tabs, W, b.reshape(1, 1))
    return out
